# baseline (speedup 1.0000x reference)
import functools

import jax
import jax.numpy as jnp
import numpy as np
from jax import lax
from jax.experimental import pallas as pl
from jax.experimental.pallas import tpu as pltpu
from jax.experimental.pallas import tpu_sc as plsc

N_ATOMS = 8192
N_EDGES = 24576
EH = N_EDGES
H = 64
HP = 128
P = 16
T_STEPS = 3

NC = 2
NS = 16
NW = NC * NS
E_PER_W = EH // NW
CHUNK = 128
NCHUNK = E_PER_W // CHUNK
STRIPE = N_ATOMS // NS


@functools.lru_cache(maxsize=None)
def _build_sc_kernels():
    mesh = plsc.VectorSubcoreMesh(
        core_axis_name="c", subcore_axis_name="s",
        num_cores=NC, num_subcores=NS)

    @functools.partial(
        pl.kernel,
        out_type=jax.ShapeDtypeStruct((EH, HP), jnp.float32),
        mesh=mesh,
        scratch_types=[
            pltpu.VMEM((NCHUNK, CHUNK), jnp.int32),
            pltpu.VMEM((E_PER_W, HP), jnp.float32),
            pltpu.SemaphoreType.DMA,
        ],
    )
    def sc_gather(table_hbm, idx_hbm, out_hbm, idx_v, rows_v, sem):
        c = lax.axis_index("c")
        s = lax.axis_index("s")
        wid = s * NC + c
        base = wid * E_PER_W
        pltpu.sync_copy(idx_hbm.at[wid], idx_v)
        copies = [
            pltpu.async_copy(table_hbm.at[idx_v.at[j]],
                             rows_v.at[pl.ds(j * CHUNK, CHUNK)], sem)
            for j in range(NCHUNK)
        ]
        for cp in copies:
            cp.wait()
        pltpu.sync_copy(rows_v, out_hbm.at[pl.ds(base, E_PER_W)])

    @functools.partial(
        pl.kernel,
        out_type=jax.ShapeDtypeStruct((NC, N_ATOMS, HP), jnp.float32),
        mesh=mesh,
        scratch_types=[
            pltpu.VMEM((NCHUNK, CHUNK), jnp.int32),
            pltpu.VMEM((NCHUNK, CHUNK), jnp.int32),
            pltpu.VMEM((2 * CHUNK, HP), jnp.float32),
            pltpu.VMEM_SHARED((N_ATOMS, HP), jnp.float32),
            pltpu.SemaphoreType.DMA,
            pltpu.SemaphoreType.DMA,
        ],
    )
    def sc_scatter_add(msg_hbm, idx_hbm, eidx_hbm, zeros_hbm,
                       out_hbm, idx_v, eidx_v, rows_v, acc, sem, sem2):
        c = lax.axis_index("c")
        s = lax.axis_index("s")
        wid = s * NC + c
        pltpu.sync_copy(zeros_hbm.at[pl.ds(s * STRIPE, STRIPE)],
                        acc.at[pl.ds(s * STRIPE, STRIPE)])
        pltpu.sync_copy(idx_hbm.at[wid], idx_v)
        pltpu.sync_copy(eidx_hbm.at[wid], eidx_v)
        plsc.subcore_barrier()
        nch = NCHUNK

        def fetch(k, sem_k):
            return pltpu.async_copy(
                msg_hbm.at[eidx_v.at[k]],
                rows_v.at[pl.ds((k % 2) * CHUNK, CHUNK)], sem_k)

        pend = fetch(0, sem)
        for k in range(nch):
            cur = pend
            if k + 1 < nch:
                nxt = fetch(k + 1, sem2 if (k + 1) % 2 else sem)
            cur.wait()
            pltpu.sync_copy(rows_v.at[pl.ds((k % 2) * CHUNK, CHUNK)],
                            acc.at[idx_v.at[k]], add=True)
            if k + 1 < nch:
                pend = nxt
        plsc.subcore_barrier()
        pltpu.sync_copy(acc.at[pl.ds(s * STRIPE, STRIPE)],
                        out_hbm.at[c, pl.ds(s * STRIPE, STRIPE)])

    return sc_gather, sc_scatter_add


def _sc_gather(table, idx3):
    return _build_sc_kernels()[0](table, idx3)


def _sc_scatter_add(msg, idx3, eidx3, zeros):
    return _build_sc_kernels()[1](msg, idx3, eidx3, zeros)


_BE = 6144


def _msg_body(g_ref, pair_ref, wt_ref, rep_ref, red_ref, b_ref, o_ref):
    f32 = jnp.float32
    g = g_ref[...]
    y = jnp.dot(g[:, :H], wt_ref[...], preferred_element_type=f32)
    pr = jnp.dot(pair_ref[...], rep_ref[...], preferred_element_type=f32)
    u = y[:, :512] * pr[:, :512] + y[:, 512:] * pr[:, 512:]
    u = u[:, :256] + u[:, 256:]
    u = u[:, :128] + u[:, 128:]
    msg = jnp.dot(u, red_ref[...], preferred_element_type=f32)
    o_ref[...] = msg + jnp.dot(g, b_ref[...], preferred_element_type=f32)


def _tc_msg(g, pair, wt, rep, red, bmat):
    return pl.pallas_call(
        _msg_body,
        grid=(EH // _BE,),
        in_specs=[
            pl.BlockSpec((_BE, HP), lambda i: (i, 0)),
            pl.BlockSpec((_BE, P), lambda i: (i, 0)),
            pl.BlockSpec((H, P * H), lambda i: (0, 0)),
            pl.BlockSpec((P, P * H), lambda i: (0, 0)),
            pl.BlockSpec((HP, HP), lambda i: (0, 0)),
            pl.BlockSpec((HP, HP), lambda i: (0, 0)),
        ],
        out_specs=pl.BlockSpec((_BE, HP), lambda i: (i, 0)),
        out_shape=jax.ShapeDtypeStruct((EH, HP), jnp.float32),
        compiler_params=pltpu.CompilerParams(
            vmem_limit_bytes=100 * 1024 * 1024),
    )(g, pair, wt, rep, red, bmat)


_BA = 2048


def _gru_compute(parts_ref, h_ref, wz_ref, wr_ref, wh_ref, uz_ref, ur_ref,
                 uh_ref, bz_ref, br_ref, bh_ref):
    m = parts_ref[0] + parts_ref[1]
    h = h_ref[...]
    f32 = jnp.float32
    z = jax.nn.sigmoid(jnp.dot(m, wz_ref[...], preferred_element_type=f32)
                       + jnp.dot(h, uz_ref[...], preferred_element_type=f32)
                       + bz_ref[...])
    r = jax.nn.sigmoid(jnp.dot(m, wr_ref[...], preferred_element_type=f32)
                       + jnp.dot(h, ur_ref[...], preferred_element_type=f32)
                       + br_ref[...])
    ht = jnp.tanh(jnp.dot(m, wh_ref[...], preferred_element_type=f32)
                  + jnp.dot(h * r, uh_ref[...], preferred_element_type=f32)
                  + bh_ref[...])
    return (1.0 - z) * ht + z * h


def _gru_body(parts_ref, h_ref, *refs):
    refs[-1][...] = _gru_compute(parts_ref, h_ref, *refs[:-1])


def _gru_body_final(parts_ref, h_ref, *refs):
    refs[-1][...] = _gru_compute(parts_ref, h_ref, *refs[:-1])[:, :H]


def _tc_gru(parts, h, wz, wr, wh, uz, ur, uh, bz, br, bh, final=False):
    full = pl.BlockSpec((HP, HP), lambda i: (0, 0))
    bias = pl.BlockSpec((1, HP), lambda i: (0, 0))
    blk = pl.BlockSpec((_BA, HP), lambda i: (i, 0))
    pblk = pl.BlockSpec((NC, _BA, HP), lambda i: (0, i, 0))
    owidth = H if final else HP
    return pl.pallas_call(
        _gru_body_final if final else _gru_body,
        grid=(N_ATOMS // _BA,),
        in_specs=[pblk, blk, full, full, full, full, full, full,
                  bias, bias, bias],
        out_specs=pl.BlockSpec((_BA, owidth), lambda i: (i, 0)),
        out_shape=jax.ShapeDtypeStruct((N_ATOMS, owidth), jnp.float32),
    )(parts, h, wz, wr, wh, uz, ur, uh, bz, br, bh)


def _pad_w(w):
    return jnp.pad(w, ((0, HP - H), (0, HP - H)))


def kernel(atom_features, pair_features, atom_to_pair, W, b,
           Wz, Wr, Wh, Uz, Ur, Uh, bz, br, bh):
    dst3 = atom_to_pair[:, 0].astype(jnp.int32).reshape(NW, NCHUNK, CHUNK)
    src3 = atom_to_pair[:, 1].astype(jnp.int32).reshape(NW, NCHUNK, CHUNK)
    eidx = jnp.arange(EH, dtype=jnp.int32).reshape(NW, NCHUNK, CHUNK)
    wt = W.reshape(P, H, H).swapaxes(0, 1).reshape(H, P * H)
    bmat = jnp.pad(b.reshape(H, H), ((0, HP - H), (0, HP - H)))
    pair16 = pair_features.astype(jnp.bfloat16)
    kk = np.arange(P * H)
    rep = jnp.asarray((kk[None, :] // H == np.arange(P)[:, None])
                      .astype(np.float32)).astype(jnp.bfloat16)
    kk2 = np.arange(HP)
    red = jnp.asarray((kk2[:, None] % H == np.arange(HP)[None, :])
                      .astype(np.float32) * (kk2[None, :] < H))
    zeros = jnp.zeros((N_ATOMS, HP), jnp.float32)
    bz2, br2, bh2 = (jnp.pad(x, (0, HP - H)).reshape(1, HP)
                     for x in (bz, br, bh))
    wz, wr, wh = _pad_w(Wz), _pad_w(Wr), _pad_w(Wh)
    uz, ur, uh = _pad_w(Uz), _pad_w(Ur), _pad_w(Uh)

    out = jnp.pad(atom_features, ((0, 0), (0, HP - H)))
    for t in range(T_STEPS):
        g = _sc_gather(out, src3)
        msg = _tc_msg(g, pair16, wt, rep, red, bmat)
        parts = _sc_scatter_add(msg, dst3, eidx, zeros)
        out = _tc_gru(parts, out, wz, wr, wh, uz, ur, uh, bz2, br2, bh2,
                      final=(t == T_STEPS - 1))
    return out


# speedup vs baseline: 1.0076x; 1.0076x over previous
import functools

import jax
import jax.numpy as jnp
import numpy as np
from jax import lax
from jax.experimental import pallas as pl
from jax.experimental.pallas import tpu as pltpu
from jax.experimental.pallas import tpu_sc as plsc

N_ATOMS = 8192
N_EDGES = 24576
EH = N_EDGES
H = 64
HP = 128
P = 16
T_STEPS = 3

NC = 2
NS = 16
NW = NC * NS
E_PER_W = EH // NW
CHUNK = 128
NCHUNK = E_PER_W // CHUNK
STRIPE = N_ATOMS // NS


@functools.lru_cache(maxsize=None)
def _build_sc_kernels():
    mesh = plsc.VectorSubcoreMesh(
        core_axis_name="c", subcore_axis_name="s",
        num_cores=NC, num_subcores=NS)

    @functools.partial(
        pl.kernel,
        out_type=jax.ShapeDtypeStruct((EH, HP), jnp.float32),
        mesh=mesh,
        scratch_types=[
            pltpu.VMEM((NCHUNK, CHUNK), jnp.int32),
            pltpu.VMEM((E_PER_W, HP), jnp.float32),
            pltpu.SemaphoreType.DMA,
        ],
    )
    def sc_gather(table_hbm, idx_hbm, out_hbm, idx_v, rows_v, sem):
        c = lax.axis_index("c")
        s = lax.axis_index("s")
        wid = s * NC + c
        base = wid * E_PER_W
        pltpu.sync_copy(idx_hbm.at[wid], idx_v)
        copies = [
            pltpu.async_copy(table_hbm.at[idx_v.at[j]],
                             rows_v.at[pl.ds(j * CHUNK, CHUNK)], sem)
            for j in range(NCHUNK)
        ]
        for cp in copies:
            cp.wait()
        pltpu.sync_copy(rows_v, out_hbm.at[pl.ds(base, E_PER_W)])

    @functools.partial(
        pl.kernel,
        out_type=jax.ShapeDtypeStruct((NC, N_ATOMS, HP), jnp.float32),
        mesh=mesh,
        scratch_types=[
            pltpu.VMEM((NCHUNK, CHUNK), jnp.int32),
            pltpu.VMEM((NCHUNK, CHUNK), jnp.int32),
            pltpu.VMEM((3 * CHUNK, HP), jnp.float32),
            pltpu.VMEM_SHARED((N_ATOMS, HP), jnp.float32),
            pltpu.SemaphoreType.DMA,
            pltpu.SemaphoreType.DMA,
            pltpu.SemaphoreType.DMA,
        ],
    )
    def sc_scatter_add(msg_hbm, idx_hbm, eidx_hbm, zeros_hbm,
                       out_hbm, idx_v, eidx_v, rows_v, acc, sem, sem2, sem3):
        c = lax.axis_index("c")
        s = lax.axis_index("s")
        wid = s * NC + c
        pltpu.sync_copy(zeros_hbm.at[pl.ds(s * STRIPE, STRIPE)],
                        acc.at[pl.ds(s * STRIPE, STRIPE)])
        pltpu.sync_copy(idx_hbm.at[wid], idx_v)
        pltpu.sync_copy(eidx_hbm.at[wid], eidx_v)
        plsc.subcore_barrier()
        nch = NCHUNK
        sems = (sem, sem2, sem3)

        def fetch(k):
            return pltpu.async_copy(
                msg_hbm.at[eidx_v.at[k]],
                rows_v.at[pl.ds((k % 3) * CHUNK, CHUNK)], sems[k % 3])

        pend = [fetch(0), fetch(1)]
        for k in range(nch):
            cur = pend.pop(0)
            if k + 2 < nch:
                pend.append(fetch(k + 2))
            cur.wait()
            pltpu.sync_copy(rows_v.at[pl.ds((k % 3) * CHUNK, CHUNK)],
                            acc.at[idx_v.at[k]], add=True)
        plsc.subcore_barrier()
        pltpu.sync_copy(acc.at[pl.ds(s * STRIPE, STRIPE)],
                        out_hbm.at[c, pl.ds(s * STRIPE, STRIPE)])

    return sc_gather, sc_scatter_add


def _sc_gather(table, idx3):
    return _build_sc_kernels()[0](table, idx3)


def _sc_scatter_add(msg, idx3, eidx3, zeros):
    return _build_sc_kernels()[1](msg, idx3, eidx3, zeros)


_BE = 4096


def _msg_body(g_ref, pair_ref, wt_ref, rep_ref, red_ref, b_ref, o_ref):
    f32 = jnp.float32
    g = g_ref[...]
    y = jnp.dot(g[:, :H], wt_ref[...], preferred_element_type=f32)
    pr = jnp.dot(pair_ref[...], rep_ref[...], preferred_element_type=f32)
    u = y[:, :512] * pr[:, :512] + y[:, 512:] * pr[:, 512:]
    u = u[:, :256] + u[:, 256:]
    u = u[:, :128] + u[:, 128:]
    msg = jnp.dot(u, red_ref[...], preferred_element_type=f32)
    o_ref[...] = msg + jnp.dot(g, b_ref[...], preferred_element_type=f32)


def _tc_msg(g, pair, wt, rep, red, bmat):
    return pl.pallas_call(
        _msg_body,
        grid=(EH // _BE,),
        in_specs=[
            pl.BlockSpec((_BE, HP), lambda i: (i, 0)),
            pl.BlockSpec((_BE, P), lambda i: (i, 0)),
            pl.BlockSpec((H, P * H), lambda i: (0, 0)),
            pl.BlockSpec((P, P * H), lambda i: (0, 0)),
            pl.BlockSpec((HP, HP), lambda i: (0, 0)),
            pl.BlockSpec((HP, HP), lambda i: (0, 0)),
        ],
        out_specs=pl.BlockSpec((_BE, HP), lambda i: (i, 0)),
        out_shape=jax.ShapeDtypeStruct((EH, HP), jnp.float32),
        compiler_params=pltpu.CompilerParams(
            vmem_limit_bytes=100 * 1024 * 1024),
    )(g, pair, wt, rep, red, bmat)


_BA = 2048


def _gru_compute(parts_ref, h_ref, wz_ref, wr_ref, wh_ref, uz_ref, ur_ref,
                 uh_ref, bz_ref, br_ref, bh_ref):
    m = parts_ref[0] + parts_ref[1]
    h = h_ref[...]
    f32 = jnp.float32
    z = jax.nn.sigmoid(jnp.dot(m, wz_ref[...], preferred_element_type=f32)
                       + jnp.dot(h, uz_ref[...], preferred_element_type=f32)
                       + bz_ref[...])
    r = jax.nn.sigmoid(jnp.dot(m, wr_ref[...], preferred_element_type=f32)
                       + jnp.dot(h, ur_ref[...], preferred_element_type=f32)
                       + br_ref[...])
    ht = jnp.tanh(jnp.dot(m, wh_ref[...], preferred_element_type=f32)
                  + jnp.dot(h * r, uh_ref[...], preferred_element_type=f32)
                  + bh_ref[...])
    return (1.0 - z) * ht + z * h


def _gru_body(parts_ref, h_ref, *refs):
    refs[-1][...] = _gru_compute(parts_ref, h_ref, *refs[:-1])


def _gru_body_final(parts_ref, h_ref, *refs):
    refs[-1][...] = _gru_compute(parts_ref, h_ref, *refs[:-1])[:, :H]


def _tc_gru(parts, h, wz, wr, wh, uz, ur, uh, bz, br, bh, final=False):
    full = pl.BlockSpec((HP, HP), lambda i: (0, 0))
    bias = pl.BlockSpec((1, HP), lambda i: (0, 0))
    blk = pl.BlockSpec((_BA, HP), lambda i: (i, 0))
    pblk = pl.BlockSpec((NC, _BA, HP), lambda i: (0, i, 0))
    owidth = H if final else HP
    return pl.pallas_call(
        _gru_body_final if final else _gru_body,
        grid=(N_ATOMS // _BA,),
        in_specs=[pblk, blk, full, full, full, full, full, full,
                  bias, bias, bias],
        out_specs=pl.BlockSpec((_BA, owidth), lambda i: (i, 0)),
        out_shape=jax.ShapeDtypeStruct((N_ATOMS, owidth), jnp.float32),
    )(parts, h, wz, wr, wh, uz, ur, uh, bz, br, bh)


def _pad_w(w):
    return jnp.pad(w, ((0, HP - H), (0, HP - H)))


def kernel(atom_features, pair_features, atom_to_pair, W, b,
           Wz, Wr, Wh, Uz, Ur, Uh, bz, br, bh):
    dst3 = atom_to_pair[:, 0].astype(jnp.int32).reshape(NW, NCHUNK, CHUNK)
    src3 = atom_to_pair[:, 1].astype(jnp.int32).reshape(NW, NCHUNK, CHUNK)
    eidx = jnp.arange(EH, dtype=jnp.int32).reshape(NW, NCHUNK, CHUNK)
    wt = W.reshape(P, H, H).swapaxes(0, 1).reshape(H, P * H)
    bmat = jnp.pad(b.reshape(H, H), ((0, HP - H), (0, HP - H)))
    pair16 = pair_features.astype(jnp.bfloat16)
    kk = np.arange(P * H)
    rep = jnp.asarray((kk[None, :] // H == np.arange(P)[:, None])
                      .astype(np.float32)).astype(jnp.bfloat16)
    kk2 = np.arange(HP)
    red = jnp.asarray((kk2[:, None] % H == np.arange(HP)[None, :])
                      .astype(np.float32) * (kk2[None, :] < H))
    zeros = jnp.zeros((N_ATOMS, HP), jnp.float32)
    bz2, br2, bh2 = (jnp.pad(x, (0, HP - H)).reshape(1, HP)
                     for x in (bz, br, bh))
    wz, wr, wh = _pad_w(Wz), _pad_w(Wr), _pad_w(Wh)
    uz, ur, uh = _pad_w(Uz), _pad_w(Ur), _pad_w(Uh)

    out = jnp.pad(atom_features, ((0, 0), (0, HP - H)))
    for t in range(T_STEPS):
        g = _sc_gather(out, src3)
        msg = _tc_msg(g, pair16, wt, rep, red, bmat)
        parts = _sc_scatter_add(msg, dst3, eidx, zeros)
        out = _tc_gru(parts, out, wz, wr, wh, uz, ur, uh, bz2, br2, bh2,
                      final=(t == T_STEPS - 1))
    return out
